# baseline (speedup 1.0000x reference)
import functools

import jax
import jax.numpy as jnp
from jax import lax
from jax.experimental import pallas as pl
from jax.experimental.pallas import tpu as pltpu
from jax.experimental.pallas import tpu_sc as plsc

SIZE = 8192
BATCH = 1024
R0, R1 = 128, 128

NC = 2
NS = 16
NW = NC * NS
BPW = BATCH // NW
CH = 2
RING = 3
DEPTH = 1

_mesh = plsc.VectorSubcoreMesh(core_axis_name="c", subcore_axis_name="s")


@functools.partial(
    pl.kernel,
    mesh=_mesh,
    out_type=jax.ShapeDtypeStruct((BATCH, R0, R1), jnp.float32),
    scratch_types=[
        pltpu.VMEM((2, 16), jnp.int32),
        pltpu.VMEM((2, 16), jnp.int32),
    ]
    + [pltpu.VMEM((CH, R0, R1), jnp.float32) for _ in range(RING)]
    + [pltpu.SemaphoreType.DMA for _ in range(2 * RING)],
)
def _sc_gather(state_hbm, keys_hbm, oprev_hbm,
               keys_v, km1_v, *bufs_and_sems):
    bufs = bufs_and_sems[:RING]
    gsem = bufs_and_sems[RING:2 * RING]
    ssem = bufs_and_sems[2 * RING:]
    wid = lax.axis_index("s") * NC + lax.axis_index("c")
    pltpu.sync_copy(keys_hbm.at[pl.ds(wid * 2, 2)], keys_v)
    for j in range(2):
        km1_v[j] = keys_v[j] - 1

    items = []
    for c in range(BPW // CH):
        j, off = divmod(c * CH, 16)
        items.append((km1_v.at[j, pl.ds(off, CH)], oprev_hbm, c * CH))
    n = len(items)

    hg = [None] * n
    hs = [None] * n

    def g_start(i):
        idx, _, _ = items[i]
        b = i % RING
        hg[i] = pltpu.async_copy(state_hbm.at[idx], bufs[b], gsem[b])

    def s_start(i):
        _, out, c = items[i]
        b = i % RING
        hs[i] = pltpu.async_copy(bufs[b], out.at[pl.ds(wid * BPW + c, CH)],
                                 ssem[b])

    for i in range(DEPTH):
        g_start(i)
    for k in range(n):
        gi = k + DEPTH
        if gi < n:
            si = gi - RING
            if si >= 0:
                hs[si].wait()
            g_start(gi)
        hg[k].wait()
        s_start(k)
    for i in range(n - RING, n):
        hs[i].wait()


KROW = 8


def _tc_rows_body(keys_smem, *refs):
    in_refs, out_ref = refs[:KROW], refs[KROW]
    for j in range(KROW):
        out_ref[pl.ds(j, 1)] = in_refs[j][...]


_tc_rows = pl.pallas_call(
    _tc_rows_body,
    grid_spec=pltpu.PrefetchScalarGridSpec(
        num_scalar_prefetch=1,
        grid=(BATCH // KROW,),
        in_specs=[
            pl.BlockSpec((1, R0, R1), (lambda i, k, j=j: (k[KROW * i + j], 0, 0)))
            for j in range(KROW)
        ],
        out_specs=pl.BlockSpec((KROW, R0, R1), lambda i, k: (i, 0, 0)),
    ),
    out_shape=jax.ShapeDtypeStruct((BATCH, R0, R1), jnp.float32),
)


def _tc_small_body(keys_ref, a_ref, r_ref, oa_ref, or_ref):
    i = pl.program_id(0)

    @pl.when(i == 0)
    def _():
        oa_ref[...] = jnp.zeros_like(oa_ref)
        or_ref[...] = jnp.zeros_like(or_ref)

    k = keys_ref[...]
    ids = lax.broadcasted_iota(jnp.int32, (BATCH, 1024), 1) + i * 1024
    oh = (k == ids).astype(jnp.float32)
    oa_ref[...] += jnp.dot(oh, a_ref[...], preferred_element_type=jnp.float32,
                           precision=lax.Precision.HIGHEST)
    or_ref[...] += jnp.dot(oh, r_ref[...], preferred_element_type=jnp.float32,
                           precision=lax.Precision.HIGHEST)


_tc_small = pl.pallas_call(
    _tc_small_body,
    grid=(SIZE // 1024,),
    in_specs=[
        pl.BlockSpec((BATCH, 1), lambda i: (0, 0)),
        pl.BlockSpec((1024, 4), lambda i: (i, 0)),
        pl.BlockSpec((1024, 1), lambda i: (i, 0)),
    ],
    out_specs=[
        pl.BlockSpec((BATCH, 4), lambda i: (0, 0)),
        pl.BlockSpec((BATCH, 1), lambda i: (0, 0)),
    ],
    out_shape=[
        jax.ShapeDtypeStruct((BATCH, 4), jnp.float32),
        jax.ShapeDtypeStruct((BATCH, 1), jnp.float32),
    ],
)


@jax.jit
def kernel(state, action, reward, keys):
    state3d = state.reshape(SIZE, R0, R1)
    keys_i32 = keys.astype(jnp.int32)
    s_prev = _sc_gather(state3d, keys_i32.reshape(64, 16))
    s_cur = _tc_rows(keys_i32, *([state3d] * KROW))
    a, r = _tc_small(keys_i32.reshape(BATCH, 1), action, reward)
    out_shape = (BATCH,) + state.shape[1:]
    return (s_prev.reshape(out_shape), a, s_cur.reshape(out_shape), r)


# speedup vs baseline: 1.6251x; 1.6251x over previous
import functools

import jax
import jax.numpy as jnp
from jax import lax
from jax.experimental import pallas as pl
from jax.experimental.pallas import tpu as pltpu
from jax.experimental.pallas import tpu_sc as plsc

SIZE = 8192
BATCH = 1024
R0, R1 = 128, 128

NC = 2
NS = 16
NW = NC * NS
BPW = BATCH // NW
CH = 2
RING = 3
DEPTH = 1

_mesh = plsc.VectorSubcoreMesh(core_axis_name="c", subcore_axis_name="s")


@functools.partial(
    pl.kernel,
    mesh=_mesh,
    out_type=[
        jax.ShapeDtypeStruct((BATCH, R0, R1), jnp.float32),
        jax.ShapeDtypeStruct((BATCH, R0, R1), jnp.float32),
        jax.ShapeDtypeStruct((BATCH, 128), jnp.float32),
        jax.ShapeDtypeStruct((BATCH, 128), jnp.float32),
    ],
    scratch_types=[
        pltpu.VMEM((2, 16), jnp.int32),
        pltpu.VMEM((2, 16), jnp.int32),
    ]
    + [pltpu.VMEM((CH, R0, R1), jnp.float32) for _ in range(RING)]
    + [
        pltpu.VMEM((2 * 16, 128), jnp.float32),
        pltpu.VMEM((2 * 16, 128), jnp.float32),
    ]
    + [pltpu.SemaphoreType.DMA for _ in range(2 * RING + 2)],
)
def _sc_gather(state_hbm, keys_hbm, act_hbm, rew_hbm,
               oprev_hbm, ocur_hbm, oact_hbm, orew_hbm,
               keys_v, km1_v, *bufs_and_sems):
    bufs = bufs_and_sems[:RING]
    abuf, rbuf = bufs_and_sems[RING:RING + 2]
    gsem = bufs_and_sems[RING + 2:2 * RING + 2]
    ssem = bufs_and_sems[2 * RING + 2:3 * RING + 2]
    arsem = bufs_and_sems[3 * RING + 2:]
    wid = lax.axis_index("s") * NC + lax.axis_index("c")
    pltpu.sync_copy(keys_hbm.at[pl.ds(wid * 2, 2)], keys_v)
    for j in range(2):
        km1_v[j] = keys_v[j] - 1

    ar_handles = []
    for j in range(2):
        ar_handles.append(pltpu.async_copy(
            act_hbm.at[keys_v.at[j]], abuf.at[pl.ds(j * 16, 16)], arsem[0]))
        ar_handles.append(pltpu.async_copy(
            rew_hbm.at[keys_v.at[j]], rbuf.at[pl.ds(j * 16, 16)], arsem[1]))

    items = []
    for c in range(BPW // CH):
        j, off = divmod(c * CH, 16)
        items.append((keys_v.at[j, pl.ds(off, CH)], ocur_hbm, c * CH))
        items.append((km1_v.at[j, pl.ds(off, CH)], oprev_hbm, c * CH))
    n = len(items)

    hg = [None] * n
    hs = [None] * n

    def g_start(i):
        idx, _, _ = items[i]
        b = i % RING
        hg[i] = pltpu.async_copy(state_hbm.at[idx], bufs[b], gsem[b])

    def s_start(i):
        _, out, c = items[i]
        b = i % RING
        hs[i] = pltpu.async_copy(bufs[b], out.at[pl.ds(wid * BPW + c, CH)],
                                 ssem[b])

    for i in range(DEPTH):
        g_start(i)
    for k in range(n):
        gi = k + DEPTH
        if gi < n:
            si = gi - RING
            if si >= 0:
                hs[si].wait()
            g_start(gi)
        hg[k].wait()
        s_start(k)
    for i in range(n - RING, n):
        hs[i].wait()

    for h in ar_handles:
        h.wait()
    pltpu.async_copy(abuf, oact_hbm.at[pl.ds(wid * BPW, BPW)],
                     arsem[0]).wait()
    pltpu.async_copy(rbuf, orew_hbm.at[pl.ds(wid * BPW, BPW)],
                     arsem[1]).wait()


@jax.jit
def kernel(state, action, reward, keys):
    state3d = state.reshape(SIZE, R0, R1)
    keys_i32 = keys.astype(jnp.int32)
    act_pad = jnp.pad(action, ((0, 0), (0, 128 - action.shape[1])))
    rew_pad = jnp.pad(reward, ((0, 0), (0, 128 - reward.shape[1])))
    s_prev, s_cur, a_pad, r_pad = _sc_gather(
        state3d, keys_i32.reshape(64, 16), act_pad, rew_pad)
    a = a_pad[:, :action.shape[1]]
    r = r_pad[:, :reward.shape[1]]
    out_shape = (BATCH,) + state.shape[1:]
    return (s_prev.reshape(out_shape), a, s_cur.reshape(out_shape), r)


# speedup vs baseline: 1.7624x; 1.0845x over previous
import functools

import jax
import jax.numpy as jnp
from jax import lax
from jax.experimental import pallas as pl
from jax.experimental.pallas import tpu as pltpu
from jax.experimental.pallas import tpu_sc as plsc

SIZE = 8192
BATCH = 1024
R0, R1 = 128, 128

NC = 2
NS = 16
NW = NC * NS
BPW = BATCH // NW
CH = 1
RING = 6
DEPTH = 2

_mesh = plsc.VectorSubcoreMesh(core_axis_name="c", subcore_axis_name="s")


@functools.partial(
    pl.kernel,
    mesh=_mesh,
    out_type=[
        jax.ShapeDtypeStruct((BATCH, R0, R1), jnp.float32),
        jax.ShapeDtypeStruct((BATCH, R0, R1), jnp.float32),
    ],
    scratch_types=[
        pltpu.VMEM((2, 16), jnp.int32),
        pltpu.VMEM((2, 16), jnp.int32),
    ]
    + [pltpu.VMEM((CH, R0, R1), jnp.float32) for _ in range(RING)]
    + [pltpu.SemaphoreType.DMA for _ in range(2 * RING)],
)
def _sc_gather(state_hbm, keys_hbm, oprev_hbm, ocur_hbm,
               keys_v, km1_v, *bufs_and_sems):
    bufs = bufs_and_sems[:RING]
    gsem = bufs_and_sems[RING:2 * RING]
    ssem = bufs_and_sems[2 * RING:]
    wid = lax.axis_index("s") * NC + lax.axis_index("c")
    pltpu.sync_copy(keys_hbm.at[pl.ds(wid * 2, 2)], keys_v)
    for j in range(2):
        km1_v[j] = keys_v[j] - 1

    items = []
    for c in range(BPW // CH):
        j, off = divmod(c * CH, 16)
        items.append((keys_v.at[j, pl.ds(off, CH)], ocur_hbm, c * CH))
        items.append((km1_v.at[j, pl.ds(off, CH)], oprev_hbm, c * CH))
    n = len(items)

    hg = [None] * n
    hs = [None] * n

    def g_start(i):
        idx, _, _ = items[i]
        b = i % RING
        hg[i] = pltpu.async_copy(state_hbm.at[idx], bufs[b], gsem[b])

    def s_start(i):
        _, out, c = items[i]
        b = i % RING
        hs[i] = pltpu.async_copy(bufs[b], out.at[pl.ds(wid * BPW + c, CH)],
                                 ssem[b])

    for i in range(DEPTH):
        g_start(i)
    for k in range(n):
        gi = k + DEPTH
        if gi < n:
            si = gi - RING
            if si >= 0:
                hs[si].wait()
            g_start(gi)
        hg[k].wait()
        s_start(k)
    for i in range(n - RING, n):
        hs[i].wait()


def _tc_small_body(keys_ref, a_ref, r_ref, oa_ref, or_ref):
    i = pl.program_id(0)

    @pl.when(i == 0)
    def _():
        oa_ref[...] = jnp.zeros_like(oa_ref)
        or_ref[...] = jnp.zeros_like(or_ref)

    k = keys_ref[...]
    ids = lax.broadcasted_iota(jnp.int32, (BATCH, 1024), 1) + i * 1024
    oh = (k == ids).astype(jnp.float32)
    oa_ref[...] += jnp.dot(oh, a_ref[...], preferred_element_type=jnp.float32,
                           precision=lax.Precision.HIGHEST)
    or_ref[...] += jnp.dot(oh, r_ref[...], preferred_element_type=jnp.float32,
                           precision=lax.Precision.HIGHEST)


_tc_small = pl.pallas_call(
    _tc_small_body,
    grid=(SIZE // 1024,),
    in_specs=[
        pl.BlockSpec((BATCH, 1), lambda i: (0, 0)),
        pl.BlockSpec((1024, 4), lambda i: (i, 0)),
        pl.BlockSpec((1024, 1), lambda i: (i, 0)),
    ],
    out_specs=[
        pl.BlockSpec((BATCH, 4), lambda i: (0, 0)),
        pl.BlockSpec((BATCH, 1), lambda i: (0, 0)),
    ],
    out_shape=[
        jax.ShapeDtypeStruct((BATCH, 4), jnp.float32),
        jax.ShapeDtypeStruct((BATCH, 1), jnp.float32),
    ],
)


@jax.jit
def kernel(state, action, reward, keys):
    state3d = state.reshape(SIZE, R0, R1)
    keys_i32 = keys.astype(jnp.int32)
    s_prev, s_cur = _sc_gather(state3d, keys_i32.reshape(64, 16))
    a, r = _tc_small(keys_i32.reshape(BATCH, 1), action, reward)
    out_shape = (BATCH,) + state.shape[1:]
    return (s_prev.reshape(out_shape), a, s_cur.reshape(out_shape), r)


# speedup vs baseline: 1.7648x; 1.0014x over previous
import functools

import jax
import jax.numpy as jnp
from jax import lax
from jax.experimental import pallas as pl
from jax.experimental.pallas import tpu as pltpu
from jax.experimental.pallas import tpu_sc as plsc

SIZE = 8192
BATCH = 1024
R0, R1 = 128, 128

NC = 2
NS = 16
NW = NC * NS
BPW = BATCH // NW
CH = 1
RING = 7
DEPTH = 2

_mesh = plsc.VectorSubcoreMesh(core_axis_name="c", subcore_axis_name="s")


@functools.partial(
    pl.kernel,
    mesh=_mesh,
    out_type=[
        jax.ShapeDtypeStruct((BATCH, R0, R1), jnp.float32),
        jax.ShapeDtypeStruct((BATCH, R0, R1), jnp.float32),
    ],
    scratch_types=[
        pltpu.VMEM((2, 16), jnp.int32),
        pltpu.VMEM((2, 16), jnp.int32),
    ]
    + [pltpu.VMEM((CH, R0, R1), jnp.float32) for _ in range(RING)]
    + [pltpu.SemaphoreType.DMA for _ in range(2 * RING)],
)
def _sc_gather(state_hbm, keys_hbm, oprev_hbm, ocur_hbm,
               keys_v, km1_v, *bufs_and_sems):
    bufs = bufs_and_sems[:RING]
    gsem = bufs_and_sems[RING:2 * RING]
    ssem = bufs_and_sems[2 * RING:]
    wid = lax.axis_index("s") * NC + lax.axis_index("c")
    pltpu.sync_copy(keys_hbm.at[pl.ds(wid * 2, 2)], keys_v)
    for j in range(2):
        km1_v[j] = keys_v[j] - 1

    items = []
    for c in range(BPW // CH):
        j, off = divmod(c * CH, 16)
        items.append((keys_v.at[j, pl.ds(off, CH)], ocur_hbm, c * CH))
        items.append((km1_v.at[j, pl.ds(off, CH)], oprev_hbm, c * CH))
    n = len(items)

    hg = [None] * n
    hs = [None] * n

    def g_start(i):
        idx, _, _ = items[i]
        b = i % RING
        hg[i] = pltpu.async_copy(state_hbm.at[idx], bufs[b], gsem[b])

    def s_start(i):
        _, out, c = items[i]
        b = i % RING
        hs[i] = pltpu.async_copy(bufs[b], out.at[pl.ds(wid * BPW + c, CH)],
                                 ssem[b])

    for i in range(DEPTH):
        g_start(i)
    for k in range(n):
        gi = k + DEPTH
        if gi < n:
            si = gi - RING
            if si >= 0:
                hs[si].wait()
            g_start(gi)
        hg[k].wait()
        s_start(k)
    for i in range(n - RING, n):
        hs[i].wait()


def _tc_small_body(keys_ref, a_ref, r_ref, oa_ref, or_ref):
    i = pl.program_id(0)

    @pl.when(i == 0)
    def _():
        oa_ref[...] = jnp.zeros_like(oa_ref)
        or_ref[...] = jnp.zeros_like(or_ref)

    k = keys_ref[...]
    ids = lax.broadcasted_iota(jnp.int32, (BATCH, 1024), 1) + i * 1024
    oh = (k == ids).astype(jnp.float32)
    oa_ref[...] += jnp.dot(oh, a_ref[...], preferred_element_type=jnp.float32,
                           precision=lax.Precision.HIGHEST)
    or_ref[...] += jnp.dot(oh, r_ref[...], preferred_element_type=jnp.float32,
                           precision=lax.Precision.HIGHEST)


_tc_small = pl.pallas_call(
    _tc_small_body,
    grid=(SIZE // 1024,),
    in_specs=[
        pl.BlockSpec((BATCH, 1), lambda i: (0, 0)),
        pl.BlockSpec((1024, 4), lambda i: (i, 0)),
        pl.BlockSpec((1024, 1), lambda i: (i, 0)),
    ],
    out_specs=[
        pl.BlockSpec((BATCH, 4), lambda i: (0, 0)),
        pl.BlockSpec((BATCH, 1), lambda i: (0, 0)),
    ],
    out_shape=[
        jax.ShapeDtypeStruct((BATCH, 4), jnp.float32),
        jax.ShapeDtypeStruct((BATCH, 1), jnp.float32),
    ],
)


@jax.jit
def kernel(state, action, reward, keys):
    state3d = state.reshape(SIZE, R0, R1)
    keys_i32 = keys.astype(jnp.int32)
    s_prev, s_cur = _sc_gather(state3d, keys_i32.reshape(64, 16))
    a, r = _tc_small(keys_i32.reshape(BATCH, 1), action, reward)
    out_shape = (BATCH,) + state.shape[1:]
    return (s_prev.reshape(out_shape), a, s_cur.reshape(out_shape), r)
